# baseline (speedup 1.0000x reference)
import functools

import jax
import jax.numpy as jnp
from jax import lax
from jax.experimental import pallas as pl
from jax.experimental.pallas import tpu as pltpu
from jax.experimental.pallas import tpu_sc as plsc

N = 10000
NP = 10240
H = 128
H2 = 64
E = 320000
NC, NS = 2, 16
CE = 80
NCHUNK = E // NS // CE
LCH = E // (NC * NS) // CE
RPT = NP // NS
ZB = 80

_mesh = plsc.VectorSubcoreMesh(
    core_axis_name="c", subcore_axis_name="s", num_cores=NC, num_subcores=NS)
_sc_params = pltpu.CompilerParams(needs_layout_passes=False,
                                  use_tc_tiling_on_sc=False)


def _agg_body(compute_cnt, table_lo, table_hi, srcs, dsts, inv_in,
              mlo_out, mhi_out, inv_out,
              sidx_v, didx_v, rows_v, rows_b, zbuf, cbuf, ones_v,
              acc_sh, cnt_sh, sem_a, sem_b):
    c = lax.axis_index("c")
    s = lax.axis_index("s")
    row0 = s * RPT

    def zrow(r, _):
        for j in range(H2 // 16):
            zbuf[r, pl.ds(j * 16, 16)] = jnp.zeros((16,), jnp.float32)
        return 0
    lax.fori_loop(0, ZB, zrow, 0)
    for j in range(CE // 16):
        ones_v[pl.ds(j * 16, 16)] = jnp.ones((16,), jnp.float32)

    def zc(j, _):
        cbuf[pl.ds(j * 16, 16)] = jnp.zeros((16,), jnp.float32)
        return 0
    lax.fori_loop(0, RPT // 16, zc, 0)

    def zero_acc():
        for kb in range(RPT // ZB):
            pltpu.sync_copy(zbuf, acc_sh.at[pl.ds(row0 + kb * ZB, ZB), :])

    pltpu.sync_copy(srcs.at[c, s], sidx_v)
    pltpu.sync_copy(dsts.at[c, s], didx_v)

    def edge_loop(table, with_cnt):
        def g_start(j, buf, sem):
            pltpu.async_copy(table.at[sidx_v.at[j]], buf, sem)

        def g_wait(j, buf, sem):
            pltpu.make_async_copy(table.at[sidx_v.at[j]], buf, sem).wait()

        def consume(j, buf):
            pltpu.sync_copy(buf, acc_sh.at[didx_v.at[j]], add=True)
            if with_cnt:
                pltpu.sync_copy(ones_v, cnt_sh.at[didx_v.at[j]], add=True)

        NPAIR = NCHUNK // 2
        g_start(0, rows_v, sem_a)

        def pair(i, _):
            i0 = 2 * i
            g_start(i0 + 1, rows_b, sem_b)
            g_wait(i0, rows_v, sem_a)
            consume(i0, rows_v)

            @pl.when(i < NPAIR - 1)
            def _():
                g_start(i0 + 2, rows_v, sem_a)
            g_wait(i0 + 1, rows_b, sem_b)
            consume(i0 + 1, rows_b)
            return 0
        lax.fori_loop(0, NPAIR, pair, 0)

    def writeout(mean_out):
        def wblk(kb, _):
            r0 = row0 + kb * ZB
            pltpu.sync_copy(acc_sh.at[pl.ds(r0, ZB), :], rows_v)

            def scale_row(r, _):
                f = plsc.load_gather(
                    cbuf, [jnp.full((16,), kb * ZB + r, jnp.int32)])
                for j in range(H2 // 16):
                    rows_v[r, pl.ds(j * 16, 16)] = (
                        rows_v[r, pl.ds(j * 16, 16)] * f)
                return 0
            lax.fori_loop(0, ZB, scale_row, 0)
            pltpu.sync_copy(rows_v, mean_out.at[c, pl.ds(r0, ZB), :])
            return 0
        lax.fori_loop(0, RPT // ZB, wblk, 0)

    zero_acc()
    if compute_cnt:
        pltpu.sync_copy(cbuf, cnt_sh.at[pl.ds(row0, RPT)])
    plsc.subcore_barrier()
    edge_loop(table_lo, compute_cnt)
    plsc.subcore_barrier()

    if compute_cnt:
        pltpu.sync_copy(cnt_sh.at[pl.ds(row0, RPT)], cbuf)

        def invb(j, _):
            v = cbuf[pl.ds(j * 16, 16)]
            cbuf[pl.ds(j * 16, 16)] = 1.0 / jnp.maximum(v, 1.0)
            return 0
        lax.fori_loop(0, RPT // 16, invb, 0)
        pltpu.sync_copy(cbuf, inv_out.at[c, pl.ds(row0, RPT)])
    else:
        pltpu.sync_copy(inv_in.at[c, pl.ds(row0, RPT)], cbuf)
    writeout(mlo_out)

    zero_acc()
    plsc.subcore_barrier()
    edge_loop(table_hi, False)
    plsc.subcore_barrier()
    writeout(mhi_out)


_agg_scratch = [
    pltpu.VMEM((NCHUNK, CE), jnp.int32),
    pltpu.VMEM((NCHUNK, CE), jnp.int32),
    pltpu.VMEM((ZB, H2), jnp.float32),
    pltpu.VMEM((ZB, H2), jnp.float32),
    pltpu.VMEM((ZB, H2), jnp.float32),
    pltpu.VMEM((RPT,), jnp.float32),
    pltpu.VMEM((CE,), jnp.float32),
    pltpu.VMEM_SHARED((NP, H2), jnp.float32),
    pltpu.VMEM_SHARED((NP,), jnp.float32),
    pltpu.SemaphoreType.DMA,
    pltpu.SemaphoreType.DMA,
]

_agg_out_type = (jax.ShapeDtypeStruct((NC, NP, H2), jnp.float32),
                 jax.ShapeDtypeStruct((NC, NP, H2), jnp.float32),
                 jax.ShapeDtypeStruct((NC, NP), jnp.float32))

_agg_first = pl.kernel(
    functools.partial(_agg_body, True),
    out_type=_agg_out_type,
    mesh=_mesh,
    scratch_types=_agg_scratch,
    compiler_params=_sc_params,
)

_agg_more = pl.kernel(
    functools.partial(_agg_body, False),
    out_type=_agg_out_type,
    mesh=_mesh,
    scratch_types=_agg_scratch,
    compiler_params=_sc_params,
)


def _classify_body(zlo, zhi, la, lb, out,
                   la_v, lb_v, bufs_a, bufs_b, ps, ol, sem_a, sem_b):
    c = lax.axis_index("c")
    s = lax.axis_index("s")
    base = (c * NS + s) * (LCH * CE)
    pltpu.sync_copy(la.at[c, s], la_v)
    pltpu.sync_copy(lb.at[c, s], lb_v)

    col_idx = [lax.iota(jnp.int32, 16) * 16 + l for l in range(16)]

    def fire(j, bufs, sem, start):
        if start:
            f = pltpu.async_copy
        else:
            f = pltpu.make_async_copy
        d0 = f(zlo.at[la_v.at[j]], bufs[0], sem)
        d1 = f(zhi.at[la_v.at[j]], bufs[1], sem)
        d2 = f(zlo.at[lb_v.at[j]], bufs[2], sem)
        d3 = f(zhi.at[lb_v.at[j]], bufs[3], sem)
        if not start:
            d0.wait(); d1.wait(); d2.wait(); d3.wait()

    def compute(i, bufs):
        alo, ahi, blo, bhi = bufs

        def grp(g, _):
            for e in range(16):
                v = alo[g * 16 + e, pl.ds(0, 16)] * blo[g * 16 + e, pl.ds(0, 16)]
                for j in range(1, H2 // 16):
                    v = v + (alo[g * 16 + e, pl.ds(j * 16, 16)]
                             * blo[g * 16 + e, pl.ds(j * 16, 16)])
                for j in range(H2 // 16):
                    v = v + (ahi[g * 16 + e, pl.ds(j * 16, 16)]
                             * bhi[g * 16 + e, pl.ds(j * 16, 16)])
                ps[pl.ds(e * 16, 16)] = v
            res = plsc.load_gather(ps, [col_idx[0]])
            for l in range(1, 16):
                res = res + plsc.load_gather(ps, [col_idx[l]])
            ol[pl.ds(i * CE + g * 16, 16)] = res
            return 0
        lax.fori_loop(0, CE // 16, grp, 0)

    NPAIR = (LCH - 1) // 2
    fire(0, bufs_a, sem_a, True)

    def pairb(i, _):
        i0 = 2 * i
        fire(i0 + 1, bufs_b, sem_b, True)
        fire(i0, bufs_a, sem_a, False)
        compute(i0, bufs_a)
        fire(i0 + 2, bufs_a, sem_a, True)
        fire(i0 + 1, bufs_b, sem_b, False)
        compute(i0 + 1, bufs_b)
        return 0
    lax.fori_loop(0, NPAIR, pairb, 0)
    fire(LCH - 1, bufs_a, sem_a, False)
    compute(LCH - 1, bufs_a)
    pltpu.sync_copy(ol, out.at[pl.ds(base, LCH * CE)])


_classify = pl.kernel(
    _classify_body,
    out_type=jax.ShapeDtypeStruct((E,), jnp.float32),
    mesh=_mesh,
    scratch_types=[
        pltpu.VMEM((LCH, CE), jnp.int32),
        pltpu.VMEM((LCH, CE), jnp.int32),
        [pltpu.VMEM((CE, H2), jnp.float32)] * 4,
        [pltpu.VMEM((CE, H2), jnp.float32)] * 4,
        pltpu.VMEM((256,), jnp.float32),
        pltpu.VMEM((LCH * CE,), jnp.float32),
        pltpu.SemaphoreType.DMA,
        pltpu.SemaphoreType.DMA,
    ],
    compiler_params=_sc_params,
)


def _tc_layer(mean_lo, mean_hi, x_lo, x_hi, wl, wr, b, relu):
    NBLK = NP // 640

    def body(mlo_ref, mhi_ref, xlo_ref, xhi_ref, wl_ref, wr_ref, b_ref,
             olo_ref, ohi_ref):
        acc = jnp.dot(mlo_ref[0], wl_ref[0, :H2, :],
                      preferred_element_type=jnp.float32)
        acc += jnp.dot(mhi_ref[0], wl_ref[0, H2:, :],
                       preferred_element_type=jnp.float32)
        acc += jnp.dot(xlo_ref[...], wr_ref[0, :H2, :],
                       preferred_element_type=jnp.float32)
        acc += jnp.dot(xhi_ref[...], wr_ref[0, H2:, :],
                       preferred_element_type=jnp.float32)
        acc += b_ref[0]
        if relu:
            acc = jnp.maximum(acc, 0.0)
        olo_ref[...] = acc[:, :H2]
        ohi_ref[...] = acc[:, H2:]

    half = pl.BlockSpec((1, 640, H2), lambda g, i: (1 - g, i, 0))
    xhalf = pl.BlockSpec((640, H2), lambda g, i: (g * NBLK + i, 0))
    wspec = pl.BlockSpec((1, H, H), lambda g, i: (g, 0, 0))
    return pl.pallas_call(
        body,
        grid=(2, NBLK),
        in_specs=[half, half, xhalf, xhalf, wspec, wspec,
                  pl.BlockSpec((1, 1, H), lambda g, i: (g, 0, 0))],
        out_specs=[pl.BlockSpec((640, H2), lambda g, i: (g * NBLK + i, 0)),
                   pl.BlockSpec((640, H2), lambda g, i: (g * NBLK + i, 0))],
        out_shape=[jax.ShapeDtypeStruct((2 * NP, H2), jnp.float32),
                   jax.ShapeDtypeStruct((2 * NP, H2), jnp.float32)],
    )(mean_lo, mean_hi, x_lo, x_hi, wl, wr, b)


def kernel(gene_node_id, disease_node_id, edge_index_gda, edge_index_rev,
           edge_label_index, gene_emb, disease_emb,
           W1_gda_l, W1_gda_r, b1_gda, W1_rev_l, W1_rev_r, b1_rev,
           W2_gda_l, W2_gda_r, b2_gda, W2_rev_l, W2_rev_r, b2_rev):
    f32, i32 = jnp.float32, jnp.int32
    xt = jnp.zeros((2 * NP, H), f32)
    xt = xt.at[:N].set(gene_emb.astype(f32))
    xt = xt.at[NP:NP + N].set(disease_emb.astype(f32))
    xt_lo, xt_hi = xt[:, :H2], xt[:, H2:]

    srcs = jnp.stack([edge_index_gda[0].astype(i32),
                      edge_index_rev[0].astype(i32) + NP]
                     ).reshape(NC, NS, NCHUNK, CE)
    dsts = jnp.stack([edge_index_gda[1].astype(i32),
                      edge_index_rev[1].astype(i32)]
                     ).reshape(NC, NS, NCHUNK, CE)
    la = edge_label_index[0].astype(i32).reshape(NC, NS, LCH, CE)
    lb = (edge_label_index[1].astype(i32) + NP).reshape(NC, NS, LCH, CE)

    dummy_inv = jnp.zeros((NC, NP), f32)
    m1_lo, m1_hi, inv = _agg_first(xt_lo, xt_hi, srcs, dsts, dummy_inv)

    wl1 = jnp.stack([W1_rev_l, W1_gda_l])
    wr1 = jnp.stack([W1_rev_r, W1_gda_r])
    bb1 = jnp.stack([b1_rev, b1_gda]).reshape(NC, 1, H)
    ht_lo, ht_hi = _tc_layer(m1_lo, m1_hi, xt_lo, xt_hi, wl1, wr1, bb1,
                             relu=True)

    m2_lo, m2_hi, _ = _agg_more(ht_lo, ht_hi, srcs, dsts, inv)

    wl2 = jnp.stack([W2_rev_l, W2_gda_l])
    wr2 = jnp.stack([W2_rev_r, W2_gda_r])
    bb2 = jnp.stack([b2_rev, b2_gda]).reshape(NC, 1, H)
    zt_lo, zt_hi = _tc_layer(m2_lo, m2_hi, ht_lo, ht_hi, wl2, wr2, bb2,
                             relu=False)

    return _classify(zt_lo, zt_hi, la, lb)


# speedup vs baseline: 1.0464x; 1.0464x over previous
import functools

import jax
import jax.numpy as jnp
from jax import lax
from jax.experimental import pallas as pl
from jax.experimental.pallas import tpu as pltpu
from jax.experimental.pallas import tpu_sc as plsc

N = 10000
NP = 10240
H = 128
H2 = 64
E = 320000
NC, NS = 2, 16
CE = 80
NCHUNK = E // NS // CE
LCH = E // (NC * NS) // CE
RPT = NP // NS
ZB = 80

_mesh = plsc.VectorSubcoreMesh(
    core_axis_name="c", subcore_axis_name="s", num_cores=NC, num_subcores=NS)
_sc_params = pltpu.CompilerParams(needs_layout_passes=False,
                                  use_tc_tiling_on_sc=False)


def _agg_body(compute_cnt, table_lo, table_hi, srcs, dsts, inv_in,
              mean_out, inv_out,
              sidx_v, didx_v, rows_v, rows_b, zbuf, cbuf, ones_v,
              acc_sh, cnt_sh, sem_a, sem_b):
    c = lax.axis_index("c")
    s = lax.axis_index("s")
    row0 = s * RPT

    def zrow(r, _):
        for j in range(H2 // 16):
            zbuf[r, pl.ds(j * 16, 16)] = jnp.zeros((16,), jnp.float32)
        return 0
    lax.fori_loop(0, ZB, zrow, 0)
    for j in range(CE // 16):
        ones_v[pl.ds(j * 16, 16)] = jnp.ones((16,), jnp.float32)

    def zc(j, _):
        cbuf[pl.ds(j * 16, 16)] = jnp.zeros((16,), jnp.float32)
        return 0
    lax.fori_loop(0, RPT // 16, zc, 0)

    def zero_acc():
        for kb in range(RPT // ZB):
            pltpu.sync_copy(zbuf, acc_sh.at[pl.ds(row0 + kb * ZB, ZB), :])

    pltpu.sync_copy(srcs.at[c, s], sidx_v)
    pltpu.sync_copy(dsts.at[c, s], didx_v)

    def edge_loop(table, with_cnt):
        def g_start(j, buf, sem):
            pltpu.async_copy(table.at[sidx_v.at[j]], buf, sem)

        def g_wait(j, buf, sem):
            pltpu.make_async_copy(table.at[sidx_v.at[j]], buf, sem).wait()

        def consume(j, buf):
            pltpu.sync_copy(buf, acc_sh.at[didx_v.at[j]], add=True)
            if with_cnt:
                pltpu.sync_copy(ones_v, cnt_sh.at[didx_v.at[j]], add=True)

        NPAIR = NCHUNK // 2
        g_start(0, rows_v, sem_a)

        def pair(i, _):
            i0 = 2 * i
            g_start(i0 + 1, rows_b, sem_b)
            g_wait(i0, rows_v, sem_a)
            consume(i0, rows_v)

            @pl.when(i < NPAIR - 1)
            def _():
                g_start(i0 + 2, rows_v, sem_a)
            g_wait(i0 + 1, rows_b, sem_b)
            consume(i0 + 1, rows_b)
            return 0
        lax.fori_loop(0, NPAIR, pair, 0)

    def writeout(col0):
        def wblk(kb, _):
            r0 = row0 + kb * ZB
            pltpu.sync_copy(acc_sh.at[pl.ds(r0, ZB), :], rows_v)

            def scale_row(r, _):
                f = plsc.load_gather(
                    cbuf, [jnp.full((16,), kb * ZB + r, jnp.int32)])
                for j in range(H2 // 16):
                    rows_v[r, pl.ds(j * 16, 16)] = (
                        rows_v[r, pl.ds(j * 16, 16)] * f)
                return 0
            lax.fori_loop(0, ZB, scale_row, 0)
            pltpu.sync_copy(
                rows_v, mean_out.at[c, pl.ds(r0, ZB), pl.ds(col0, H2)])
            return 0
        lax.fori_loop(0, RPT // ZB, wblk, 0)

    zero_acc()
    if compute_cnt:
        pltpu.sync_copy(cbuf, cnt_sh.at[pl.ds(row0, RPT)])
    plsc.subcore_barrier()
    edge_loop(table_lo, compute_cnt)
    plsc.subcore_barrier()

    if compute_cnt:
        pltpu.sync_copy(cnt_sh.at[pl.ds(row0, RPT)], cbuf)

        def invb(j, _):
            v = cbuf[pl.ds(j * 16, 16)]
            cbuf[pl.ds(j * 16, 16)] = 1.0 / jnp.maximum(v, 1.0)
            return 0
        lax.fori_loop(0, RPT // 16, invb, 0)
        pltpu.sync_copy(cbuf, inv_out.at[c, pl.ds(row0, RPT)])
    else:
        pltpu.sync_copy(inv_in.at[c, pl.ds(row0, RPT)], cbuf)
    writeout(0)

    zero_acc()
    plsc.subcore_barrier()
    edge_loop(table_hi, False)
    plsc.subcore_barrier()
    writeout(H2)


_agg_scratch = [
    pltpu.VMEM((NCHUNK, CE), jnp.int32),
    pltpu.VMEM((NCHUNK, CE), jnp.int32),
    pltpu.VMEM((ZB, H2), jnp.float32),
    pltpu.VMEM((ZB, H2), jnp.float32),
    pltpu.VMEM((ZB, H2), jnp.float32),
    pltpu.VMEM((RPT,), jnp.float32),
    pltpu.VMEM((CE,), jnp.float32),
    pltpu.VMEM_SHARED((NP, H2), jnp.float32),
    pltpu.VMEM_SHARED((NP,), jnp.float32),
    pltpu.SemaphoreType.DMA,
    pltpu.SemaphoreType.DMA,
]

_agg_out_type = (jax.ShapeDtypeStruct((NC, NP, H), jnp.float32),
                 jax.ShapeDtypeStruct((NC, NP), jnp.float32))

_agg_first = pl.kernel(
    functools.partial(_agg_body, True),
    out_type=_agg_out_type,
    mesh=_mesh,
    scratch_types=_agg_scratch,
    compiler_params=_sc_params,
)

_agg_more = pl.kernel(
    functools.partial(_agg_body, False),
    out_type=_agg_out_type,
    mesh=_mesh,
    scratch_types=_agg_scratch,
    compiler_params=_sc_params,
)


def _classify_body(z, la, lb, out, la_v, lb_v, bufs_a, bufs_b, ps, ol,
                   sem_a, sem_b):
    c = lax.axis_index("c")
    s = lax.axis_index("s")
    base = (c * NS + s) * (LCH * CE)
    pltpu.sync_copy(la.at[c, s], la_v)
    pltpu.sync_copy(lb.at[c, s], lb_v)

    col_idx = [lax.iota(jnp.int32, 16) * 16 + l for l in range(16)]

    def fire(j, bufs, sem, start):
        if start:
            f = pltpu.async_copy
        else:
            f = pltpu.make_async_copy
        d0 = f(z.at[la_v.at[j]], bufs[0], sem)
        d1 = f(z.at[lb_v.at[j]], bufs[1], sem)
        if not start:
            d0.wait()
            d1.wait()

    def compute(i, bufs):
        av, bv = bufs

        def grp(g, _):
            for e in range(16):
                v = av[g * 16 + e, pl.ds(0, 16)] * bv[g * 16 + e, pl.ds(0, 16)]
                for j in range(1, H // 16):
                    v = v + (av[g * 16 + e, pl.ds(j * 16, 16)]
                             * bv[g * 16 + e, pl.ds(j * 16, 16)])
                ps[pl.ds(e * 16, 16)] = v
            res = plsc.load_gather(ps, [col_idx[0]])
            for l in range(1, 16):
                res = res + plsc.load_gather(ps, [col_idx[l]])
            ol[pl.ds(i * CE + g * 16, 16)] = res
            return 0
        lax.fori_loop(0, CE // 16, grp, 0)

    NPAIR = (LCH - 1) // 2
    fire(0, bufs_a, sem_a, True)

    def pairb(i, _):
        i0 = 2 * i
        fire(i0 + 1, bufs_b, sem_b, True)
        fire(i0, bufs_a, sem_a, False)
        compute(i0, bufs_a)
        fire(i0 + 2, bufs_a, sem_a, True)
        fire(i0 + 1, bufs_b, sem_b, False)
        compute(i0 + 1, bufs_b)
        return 0
    lax.fori_loop(0, NPAIR, pairb, 0)
    fire(LCH - 1, bufs_a, sem_a, False)
    compute(LCH - 1, bufs_a)
    pltpu.sync_copy(ol, out.at[pl.ds(base, LCH * CE)])


_classify = pl.kernel(
    _classify_body,
    out_type=jax.ShapeDtypeStruct((E,), jnp.float32),
    mesh=_mesh,
    scratch_types=[
        pltpu.VMEM((LCH, CE), jnp.int32),
        pltpu.VMEM((LCH, CE), jnp.int32),
        [pltpu.VMEM((CE, H), jnp.float32)] * 2,
        [pltpu.VMEM((CE, H), jnp.float32)] * 2,
        pltpu.VMEM((256,), jnp.float32),
        pltpu.VMEM((LCH * CE,), jnp.float32),
        pltpu.SemaphoreType.DMA,
        pltpu.SemaphoreType.DMA,
    ],
    compiler_params=_sc_params,
)


def _tc_layer(mean, x_lo, x_hi, wl, wr, b, relu, split_out):
    NBLK = NP // 640

    def body(mean_ref, xlo_ref, xhi_ref, wl_ref, wr_ref, b_ref, *outs):
        acc = jnp.dot(mean_ref[0], wl_ref[0],
                      preferred_element_type=jnp.float32)
        acc += jnp.dot(xlo_ref[...], wr_ref[0, :H2, :],
                       preferred_element_type=jnp.float32)
        acc += jnp.dot(xhi_ref[...], wr_ref[0, H2:, :],
                       preferred_element_type=jnp.float32)
        acc += b_ref[0]
        if relu:
            acc = jnp.maximum(acc, 0.0)
        if split_out:
            outs[0][...] = acc[:, :H2]
            outs[1][...] = acc[:, H2:]
        else:
            outs[0][...] = acc

    wspec = pl.BlockSpec((1, H, H), lambda g, i: (g, 0, 0))
    xhalf = pl.BlockSpec((640, H2), lambda g, i: (g * NBLK + i, 0))
    ohalf = pl.BlockSpec((640, H2), lambda g, i: (g * NBLK + i, 0))
    if split_out:
        out_specs = [ohalf, ohalf]
        out_shape = [jax.ShapeDtypeStruct((2 * NP, H2), jnp.float32),
                     jax.ShapeDtypeStruct((2 * NP, H2), jnp.float32)]
    else:
        out_specs = [pl.BlockSpec((640, H), lambda g, i: (g * NBLK + i, 0))]
        out_shape = [jax.ShapeDtypeStruct((2 * NP, H), jnp.float32)]
    return pl.pallas_call(
        body,
        grid=(2, NBLK),
        in_specs=[pl.BlockSpec((1, 640, H), lambda g, i: (1 - g, i, 0)),
                  xhalf, xhalf, wspec, wspec,
                  pl.BlockSpec((1, 1, H), lambda g, i: (g, 0, 0))],
        out_specs=out_specs,
        out_shape=out_shape,
    )(mean, x_lo, x_hi, wl, wr, b)


def kernel(gene_node_id, disease_node_id, edge_index_gda, edge_index_rev,
           edge_label_index, gene_emb, disease_emb,
           W1_gda_l, W1_gda_r, b1_gda, W1_rev_l, W1_rev_r, b1_rev,
           W2_gda_l, W2_gda_r, b2_gda, W2_rev_l, W2_rev_r, b2_rev):
    f32, i32 = jnp.float32, jnp.int32
    xt_lo = jnp.zeros((2 * NP, H2), f32)
    xt_lo = xt_lo.at[:N].set(gene_emb[:, :H2].astype(f32))
    xt_lo = xt_lo.at[NP:NP + N].set(disease_emb[:, :H2].astype(f32))
    xt_hi = jnp.zeros((2 * NP, H2), f32)
    xt_hi = xt_hi.at[:N].set(gene_emb[:, H2:].astype(f32))
    xt_hi = xt_hi.at[NP:NP + N].set(disease_emb[:, H2:].astype(f32))

    srcs = jnp.stack([edge_index_gda[0].astype(i32),
                      edge_index_rev[0].astype(i32) + NP]
                     ).reshape(NC, NS, NCHUNK, CE)
    dsts = jnp.stack([edge_index_gda[1].astype(i32),
                      edge_index_rev[1].astype(i32)]
                     ).reshape(NC, NS, NCHUNK, CE)
    la = edge_label_index[0].astype(i32).reshape(NC, NS, LCH, CE)
    lb = (edge_label_index[1].astype(i32) + NP).reshape(NC, NS, LCH, CE)

    dummy_inv = jnp.zeros((NC, NP), f32)
    m1, inv = _agg_first(xt_lo, xt_hi, srcs, dsts, dummy_inv)

    wl1 = jnp.stack([W1_rev_l, W1_gda_l])
    wr1 = jnp.stack([W1_rev_r, W1_gda_r])
    bb1 = jnp.stack([b1_rev, b1_gda]).reshape(NC, 1, H)
    ht_lo, ht_hi = _tc_layer(m1, xt_lo, xt_hi, wl1, wr1, bb1,
                             relu=True, split_out=True)

    m2, _ = _agg_more(ht_lo, ht_hi, srcs, dsts, inv)

    wl2 = jnp.stack([W2_rev_l, W2_gda_l])
    wr2 = jnp.stack([W2_rev_r, W2_gda_r])
    bb2 = jnp.stack([b2_rev, b2_gda]).reshape(NC, 1, H)
    (zt,) = _tc_layer(m2, ht_lo, ht_hi, wl2, wr2, bb2,
                      relu=False, split_out=False)

    return _classify(zt, la, lb)


# speedup vs baseline: 1.3969x; 1.3349x over previous
import functools

import jax
import jax.numpy as jnp
from jax import lax
from jax.experimental import pallas as pl
from jax.experimental.pallas import tpu as pltpu
from jax.experimental.pallas import tpu_sc as plsc

N = 10000
NP = 10240
H = 128
H2 = 64
E = 320000
NC, NS = 2, 16
CE = 125
NCHUNK = E // NS // CE
NQUAD = NCHUNK // 4
CL = 80
LCH = E // (NC * NS) // CL
RPT = NP // NS
ZB = 80

_mesh = plsc.VectorSubcoreMesh(
    core_axis_name="c", subcore_axis_name="s", num_cores=NC, num_subcores=NS)
_sc_params = pltpu.CompilerParams(needs_layout_passes=False,
                                  use_tc_tiling_on_sc=False)


def _agg_body(compute_cnt, table_lo, table_hi, srcs, dsts, inv_in,
              mean_out, inv_out,
              sidx_v, didx_v, bufs, zbuf, cbuf, ones_v,
              acc_sh, cnt_sh, gsems, ssems):
    c = lax.axis_index("c")
    s = lax.axis_index("s")
    row0 = s * RPT

    def zrow(r, _):
        for j in range(H2 // 16):
            zbuf[r, pl.ds(j * 16, 16)] = jnp.zeros((16,), jnp.float32)
        return 0
    lax.fori_loop(0, ZB, zrow, 0)
    for j in range(128 // 16):
        ones_v[pl.ds(j * 16, 16)] = jnp.ones((16,), jnp.float32)

    def zc(j, _):
        cbuf[pl.ds(j * 16, 16)] = jnp.zeros((16,), jnp.float32)
        return 0
    lax.fori_loop(0, RPT // 16, zc, 0)

    def zero_acc():
        for kb in range(RPT // ZB):
            pltpu.sync_copy(zbuf, acc_sh.at[pl.ds(row0 + kb * ZB, ZB), :])

    pltpu.sync_copy(srcs.at[c, s], sidx_v)
    pltpu.sync_copy(dsts.at[c, s], didx_v)

    def edge_loop(table, with_cnt):
        def g_start(j, k):
            pltpu.async_copy(table.at[sidx_v.at[j]], bufs[k], gsems[k])

        def g_wait(j, k):
            pltpu.make_async_copy(
                table.at[sidx_v.at[j]], bufs[k], gsems[k]).wait()

        def s_start(j, k):
            pltpu.async_copy(bufs[k], acc_sh.at[didx_v.at[j]], ssems[k],
                             add=True)

        def s_wait(j, k):
            pltpu.make_async_copy(
                bufs[k], acc_sh.at[didx_v.at[j]], ssems[k]).wait()

        g_start(0, 0)
        g_start(1, 1)
        g_start(2, 2)

        def quad(q, _):
            j0 = 4 * q
            for k in range(4):
                j = j0 + k
                g_wait(j, k)
                kp = (k + 3) % 4
                if k == 0:
                    @pl.when(q > 0)
                    def _():
                        s_wait(j - 1, kp)
                else:
                    s_wait(j - 1, kp)
                s_start(j, k)
                if k == 0:
                    g_start(j + 3, kp)
                else:
                    @pl.when(q < NQUAD - 1)
                    def _():
                        g_start(j + 3, kp)
                if with_cnt:
                    pltpu.sync_copy(ones_v.at[pl.ds(0, CE)],
                                    cnt_sh.at[didx_v.at[j]], add=True)
            return 0
        lax.fori_loop(0, NQUAD, quad, 0)
        s_wait(NCHUNK - 1, 3)

    def writeout(col0):
        wv = bufs[0]

        def wblk(kb, _):
            r0 = row0 + kb * ZB
            pltpu.sync_copy(acc_sh.at[pl.ds(r0, ZB), :],
                            wv.at[pl.ds(0, ZB), :])

            def scale_row(r, _):
                f = plsc.load_gather(
                    cbuf, [jnp.full((16,), kb * ZB + r, jnp.int32)])
                for j in range(H2 // 16):
                    wv[r, pl.ds(j * 16, 16)] = wv[r, pl.ds(j * 16, 16)] * f
                return 0
            lax.fori_loop(0, ZB, scale_row, 0)
            pltpu.sync_copy(
                wv.at[pl.ds(0, ZB), :],
                mean_out.at[c, pl.ds(r0, ZB), pl.ds(col0, H2)])
            return 0
        lax.fori_loop(0, RPT // ZB, wblk, 0)

    zero_acc()
    if compute_cnt:
        pltpu.sync_copy(cbuf, cnt_sh.at[pl.ds(row0, RPT)])
    plsc.subcore_barrier()
    edge_loop(table_lo, compute_cnt)
    plsc.subcore_barrier()

    if compute_cnt:
        pltpu.sync_copy(cnt_sh.at[pl.ds(row0, RPT)], cbuf)

        def invb(j, _):
            v = cbuf[pl.ds(j * 16, 16)]
            cbuf[pl.ds(j * 16, 16)] = 1.0 / jnp.maximum(v, 1.0)
            return 0
        lax.fori_loop(0, RPT // 16, invb, 0)
        pltpu.sync_copy(cbuf, inv_out.at[c, pl.ds(row0, RPT)])
    else:
        pltpu.sync_copy(inv_in.at[c, pl.ds(row0, RPT)], cbuf)
    writeout(0)

    zero_acc()
    plsc.subcore_barrier()
    edge_loop(table_hi, False)
    plsc.subcore_barrier()
    writeout(H2)


_agg_scratch = [
    pltpu.VMEM((NCHUNK, CE), jnp.int32),
    pltpu.VMEM((NCHUNK, CE), jnp.int32),
    [pltpu.VMEM((CE, H2), jnp.float32)] * 4,
    pltpu.VMEM((ZB, H2), jnp.float32),
    pltpu.VMEM((RPT,), jnp.float32),
    pltpu.VMEM((128,), jnp.float32),
    pltpu.VMEM_SHARED((NP, H2), jnp.float32),
    pltpu.VMEM_SHARED((NP,), jnp.float32),
    [pltpu.SemaphoreType.DMA] * 4,
    [pltpu.SemaphoreType.DMA] * 4,
]

_agg_out_type = (jax.ShapeDtypeStruct((NC, NP, H), jnp.float32),
                 jax.ShapeDtypeStruct((NC, NP), jnp.float32))

_agg_first = pl.kernel(
    functools.partial(_agg_body, True),
    out_type=_agg_out_type,
    mesh=_mesh,
    scratch_types=_agg_scratch,
    compiler_params=_sc_params,
)

_agg_more = pl.kernel(
    functools.partial(_agg_body, False),
    out_type=_agg_out_type,
    mesh=_mesh,
    scratch_types=_agg_scratch,
    compiler_params=_sc_params,
)


def _classify_body(z, la, lb, out, la_v, lb_v, bufs_a, bufs_b, ps, ol,
                   sem_a, sem_b):
    c = lax.axis_index("c")
    s = lax.axis_index("s")
    base = (c * NS + s) * (LCH * CL)
    pltpu.sync_copy(la.at[c, s], la_v)
    pltpu.sync_copy(lb.at[c, s], lb_v)

    col_idx = [lax.iota(jnp.int32, 16) * 16 + l for l in range(16)]

    def fire(j, bufs, sem, start):
        if start:
            f = pltpu.async_copy
        else:
            f = pltpu.make_async_copy
        d0 = f(z.at[la_v.at[j]], bufs[0], sem)
        d1 = f(z.at[lb_v.at[j]], bufs[1], sem)
        if not start:
            d0.wait()
            d1.wait()

    def compute(i, bufs):
        av, bv = bufs

        def grp(g, _):
            for e in range(16):
                v = av[g * 16 + e, pl.ds(0, 16)] * bv[g * 16 + e, pl.ds(0, 16)]
                for j in range(1, H // 16):
                    v = v + (av[g * 16 + e, pl.ds(j * 16, 16)]
                             * bv[g * 16 + e, pl.ds(j * 16, 16)])
                ps[pl.ds(e * 16, 16)] = v
            res = plsc.load_gather(ps, [col_idx[0]])
            for l in range(1, 16):
                res = res + plsc.load_gather(ps, [col_idx[l]])
            ol[pl.ds(i * CL + g * 16, 16)] = res
            return 0
        lax.fori_loop(0, CL // 16, grp, 0)

    NPAIR = (LCH - 1) // 2
    fire(0, bufs_a, sem_a, True)

    def pairb(i, _):
        i0 = 2 * i
        fire(i0 + 1, bufs_b, sem_b, True)
        fire(i0, bufs_a, sem_a, False)
        compute(i0, bufs_a)
        fire(i0 + 2, bufs_a, sem_a, True)
        fire(i0 + 1, bufs_b, sem_b, False)
        compute(i0 + 1, bufs_b)
        return 0
    lax.fori_loop(0, NPAIR, pairb, 0)
    fire(LCH - 1, bufs_a, sem_a, False)
    compute(LCH - 1, bufs_a)
    pltpu.sync_copy(ol, out.at[pl.ds(base, LCH * CL)])


_classify = pl.kernel(
    _classify_body,
    out_type=jax.ShapeDtypeStruct((E,), jnp.float32),
    mesh=_mesh,
    scratch_types=[
        pltpu.VMEM((LCH, CL), jnp.int32),
        pltpu.VMEM((LCH, CL), jnp.int32),
        [pltpu.VMEM((CL, H), jnp.float32)] * 2,
        [pltpu.VMEM((CL, H), jnp.float32)] * 2,
        pltpu.VMEM((256,), jnp.float32),
        pltpu.VMEM((LCH * CL,), jnp.float32),
        pltpu.SemaphoreType.DMA,
        pltpu.SemaphoreType.DMA,
    ],
    compiler_params=_sc_params,
)


def _tc_layer(mean, x_lo, x_hi, wl, wr, b, relu, split_out):
    NBLK = NP // 640

    def body(mean_ref, xlo_ref, xhi_ref, wl_ref, wr_ref, b_ref, *outs):
        acc = jnp.dot(mean_ref[0], wl_ref[0],
                      preferred_element_type=jnp.float32)
        acc += jnp.dot(xlo_ref[...], wr_ref[0, :H2, :],
                       preferred_element_type=jnp.float32)
        acc += jnp.dot(xhi_ref[...], wr_ref[0, H2:, :],
                       preferred_element_type=jnp.float32)
        acc += b_ref[0]
        if relu:
            acc = jnp.maximum(acc, 0.0)
        if split_out:
            outs[0][...] = acc[:, :H2]
            outs[1][...] = acc[:, H2:]
        else:
            outs[0][...] = acc

    wspec = pl.BlockSpec((1, H, H), lambda g, i: (g, 0, 0))
    xhalf = pl.BlockSpec((640, H2), lambda g, i: (g * NBLK + i, 0))
    ohalf = pl.BlockSpec((640, H2), lambda g, i: (g * NBLK + i, 0))
    if split_out:
        out_specs = [ohalf, ohalf]
        out_shape = [jax.ShapeDtypeStruct((2 * NP, H2), jnp.float32),
                     jax.ShapeDtypeStruct((2 * NP, H2), jnp.float32)]
    else:
        out_specs = [pl.BlockSpec((640, H), lambda g, i: (g * NBLK + i, 0))]
        out_shape = [jax.ShapeDtypeStruct((2 * NP, H), jnp.float32)]
    return pl.pallas_call(
        body,
        grid=(2, NBLK),
        in_specs=[pl.BlockSpec((1, 640, H), lambda g, i: (1 - g, i, 0)),
                  xhalf, xhalf, wspec, wspec,
                  pl.BlockSpec((1, 1, H), lambda g, i: (g, 0, 0))],
        out_specs=out_specs,
        out_shape=out_shape,
    )(mean, x_lo, x_hi, wl, wr, b)


def kernel(gene_node_id, disease_node_id, edge_index_gda, edge_index_rev,
           edge_label_index, gene_emb, disease_emb,
           W1_gda_l, W1_gda_r, b1_gda, W1_rev_l, W1_rev_r, b1_rev,
           W2_gda_l, W2_gda_r, b2_gda, W2_rev_l, W2_rev_r, b2_rev):
    f32, i32 = jnp.float32, jnp.int32
    xt_lo = jnp.zeros((2 * NP, H2), f32)
    xt_lo = xt_lo.at[:N].set(gene_emb[:, :H2].astype(f32))
    xt_lo = xt_lo.at[NP:NP + N].set(disease_emb[:, :H2].astype(f32))
    xt_hi = jnp.zeros((2 * NP, H2), f32)
    xt_hi = xt_hi.at[:N].set(gene_emb[:, H2:].astype(f32))
    xt_hi = xt_hi.at[NP:NP + N].set(disease_emb[:, H2:].astype(f32))

    srcs = jnp.stack([edge_index_gda[0].astype(i32),
                      edge_index_rev[0].astype(i32) + NP]
                     ).reshape(NC, NS, NCHUNK, CE)
    dsts = jnp.stack([edge_index_gda[1].astype(i32),
                      edge_index_rev[1].astype(i32)]
                     ).reshape(NC, NS, NCHUNK, CE)
    la = edge_label_index[0].astype(i32).reshape(NC, NS, LCH, CL)
    lb = (edge_label_index[1].astype(i32) + NP).reshape(NC, NS, LCH, CL)

    dummy_inv = jnp.zeros((NC, NP), f32)
    m1, inv = _agg_first(xt_lo, xt_hi, srcs, dsts, dummy_inv)

    wl1 = jnp.stack([W1_rev_l, W1_gda_l])
    wr1 = jnp.stack([W1_rev_r, W1_gda_r])
    bb1 = jnp.stack([b1_rev, b1_gda]).reshape(NC, 1, H)
    ht_lo, ht_hi = _tc_layer(m1, xt_lo, xt_hi, wl1, wr1, bb1,
                             relu=True, split_out=True)

    m2, _ = _agg_more(ht_lo, ht_hi, srcs, dsts, inv)

    wl2 = jnp.stack([W2_rev_l, W2_gda_l])
    wr2 = jnp.stack([W2_rev_r, W2_gda_r])
    bb2 = jnp.stack([b2_rev, b2_gda]).reshape(NC, 1, H)
    (zt,) = _tc_layer(m2, ht_lo, ht_hi, wl2, wr2, bb2,
                      relu=False, split_out=False)

    return _classify(zt, la, lb)


# speedup vs baseline: 1.4675x; 1.0505x over previous
import functools

import jax
import jax.numpy as jnp
from jax import lax
from jax.experimental import pallas as pl
from jax.experimental.pallas import tpu as pltpu
from jax.experimental.pallas import tpu_sc as plsc

N = 10000
NP = 10240
H = 128
H2 = 64
E = 320000
NC, NS = 2, 16
CE = 125
NCHUNK = E // NS // CE
NQUAD = NCHUNK // 4
CL = 80
LCH = E // (NC * NS) // CL
RPT = NP // NS
ZB = 80

_mesh = plsc.VectorSubcoreMesh(
    core_axis_name="c", subcore_axis_name="s", num_cores=NC, num_subcores=NS)
_sc_params = pltpu.CompilerParams(needs_layout_passes=False,
                                  use_tc_tiling_on_sc=False)


def _agg_body(compute_cnt, g_lo, g_hi, d_lo, d_hi, eg, er, inv_in,
              mean_out, inv_out,
              sidx_v, didx_v, bufs, zbuf, cbuf, ones_v,
              acc_sh, cnt_sh, gsems, ssems):
    c = lax.axis_index("c")
    s = lax.axis_index("s")
    row0 = s * RPT

    def zrow(r, _):
        for j in range(H2 // 16):
            zbuf[r, pl.ds(j * 16, 16)] = jnp.zeros((16,), jnp.float32)
        return 0
    lax.fori_loop(0, ZB, zrow, 0)
    for j in range(128 // 16):
        ones_v[pl.ds(j * 16, 16)] = jnp.ones((16,), jnp.float32)

    def zc(j, _):
        cbuf[pl.ds(j * 16, 16)] = jnp.zeros((16,), jnp.float32)
        return 0
    lax.fori_loop(0, RPT // 16, zc, 0)

    def zero_acc():
        for kb in range(RPT // ZB):
            pltpu.sync_copy(zbuf, acc_sh.at[pl.ds(row0 + kb * ZB, ZB), :])

    @pl.when(c == 0)
    def _():
        pltpu.sync_copy(eg.at[0, s], sidx_v)
        pltpu.sync_copy(eg.at[1, s], didx_v)

    @pl.when(c == 1)
    def _():
        pltpu.sync_copy(er.at[0, s], sidx_v)
        pltpu.sync_copy(er.at[1, s], didx_v)

    def edge_loop(table, with_cnt):
        def g_start(j, k):
            pltpu.async_copy(table.at[sidx_v.at[j]], bufs[k], gsems[k])

        def g_wait(j, k):
            pltpu.make_async_copy(
                table.at[sidx_v.at[j]], bufs[k], gsems[k]).wait()

        def s_start(j, k):
            pltpu.async_copy(bufs[k], acc_sh.at[didx_v.at[j]], ssems[k],
                             add=True)

        def s_wait(j, k):
            pltpu.make_async_copy(
                bufs[k], acc_sh.at[didx_v.at[j]], ssems[k]).wait()

        g_start(0, 0)
        g_start(1, 1)
        g_start(2, 2)

        def quad(q, _):
            j0 = 4 * q
            for k in range(4):
                j = j0 + k
                g_wait(j, k)
                kp = (k + 3) % 4
                if k == 0:
                    @pl.when(q > 0)
                    def _():
                        s_wait(j - 1, kp)
                else:
                    s_wait(j - 1, kp)
                s_start(j, k)
                if k == 0:
                    g_start(j + 3, kp)
                else:
                    @pl.when(q < NQUAD - 1)
                    def _():
                        g_start(j + 3, kp)
                if with_cnt:
                    pltpu.sync_copy(ones_v.at[pl.ds(0, CE)],
                                    cnt_sh.at[didx_v.at[j]], add=True)
            return 0
        lax.fori_loop(0, NQUAD, quad, 0)
        s_wait(NCHUNK - 1, 3)

    def writeout(col0):
        wv = bufs[0]

        def wblk(kb, _):
            r0 = row0 + kb * ZB
            pltpu.sync_copy(acc_sh.at[pl.ds(r0, ZB), :],
                            wv.at[pl.ds(0, ZB), :])

            def scale_row(r, _):
                f = plsc.load_gather(
                    cbuf, [jnp.full((16,), kb * ZB + r, jnp.int32)])
                for j in range(H2 // 16):
                    wv[r, pl.ds(j * 16, 16)] = wv[r, pl.ds(j * 16, 16)] * f
                return 0
            lax.fori_loop(0, ZB, scale_row, 0)
            pltpu.sync_copy(
                wv.at[pl.ds(0, ZB), :],
                mean_out.at[c, pl.ds(r0, ZB), pl.ds(col0, H2)])
            return 0
        lax.fori_loop(0, RPT // ZB, wblk, 0)

    zero_acc()
    if compute_cnt:
        pltpu.sync_copy(cbuf, cnt_sh.at[pl.ds(row0, RPT)])
    plsc.subcore_barrier()

    @pl.when(c == 0)
    def _():
        edge_loop(g_lo, compute_cnt)

    @pl.when(c == 1)
    def _():
        edge_loop(d_lo, compute_cnt)
    plsc.subcore_barrier()

    if compute_cnt:
        pltpu.sync_copy(cnt_sh.at[pl.ds(row0, RPT)], cbuf)

        def invb(j, _):
            v = cbuf[pl.ds(j * 16, 16)]
            cbuf[pl.ds(j * 16, 16)] = 1.0 / jnp.maximum(v, 1.0)
            return 0
        lax.fori_loop(0, RPT // 16, invb, 0)
        pltpu.sync_copy(cbuf, inv_out.at[c, pl.ds(row0, RPT)])
    else:
        pltpu.sync_copy(inv_in.at[c, pl.ds(row0, RPT)], cbuf)
    writeout(0)

    zero_acc()
    plsc.subcore_barrier()

    @pl.when(c == 0)
    def _():
        edge_loop(g_hi, False)

    @pl.when(c == 1)
    def _():
        edge_loop(d_hi, False)
    plsc.subcore_barrier()
    writeout(H2)


_agg_scratch = [
    pltpu.VMEM((NCHUNK, CE), jnp.int32),
    pltpu.VMEM((NCHUNK, CE), jnp.int32),
    [pltpu.VMEM((CE, H2), jnp.float32)] * 4,
    pltpu.VMEM((ZB, H2), jnp.float32),
    pltpu.VMEM((RPT,), jnp.float32),
    pltpu.VMEM((128,), jnp.float32),
    pltpu.VMEM_SHARED((NP, H2), jnp.float32),
    pltpu.VMEM_SHARED((NP,), jnp.float32),
    [pltpu.SemaphoreType.DMA] * 4,
    [pltpu.SemaphoreType.DMA] * 4,
]

_agg_out_type = (jax.ShapeDtypeStruct((NC, NP, H), jnp.float32),
                 jax.ShapeDtypeStruct((NC, NP), jnp.float32))

_agg_first = pl.kernel(
    functools.partial(_agg_body, True),
    out_type=_agg_out_type,
    mesh=_mesh,
    scratch_types=_agg_scratch,
    compiler_params=_sc_params,
)

_agg_more = pl.kernel(
    functools.partial(_agg_body, False),
    out_type=_agg_out_type,
    mesh=_mesh,
    scratch_types=_agg_scratch,
    compiler_params=_sc_params,
)


def _classify_body(zg, zd, el, out, la_v, lb_v, bufs_a, bufs_b, ps, ol,
                   sem_a, sem_b):
    c = lax.axis_index("c")
    s = lax.axis_index("s")
    base = (c * NS + s) * (LCH * CL)
    pltpu.sync_copy(el.at[0, c, s], la_v)
    pltpu.sync_copy(el.at[1, c, s], lb_v)

    col_idx = [lax.iota(jnp.int32, 16) * 16 + l for l in range(16)]

    def fire(j, bufs, sem, start):
        if start:
            f = pltpu.async_copy
        else:
            f = pltpu.make_async_copy
        d0 = f(zg.at[la_v.at[j]], bufs[0], sem)
        d1 = f(zd.at[lb_v.at[j]], bufs[1], sem)
        if not start:
            d0.wait()
            d1.wait()

    def compute(i, bufs):
        av, bv = bufs

        def grp(g, _):
            for e in range(16):
                v = av[g * 16 + e, pl.ds(0, 16)] * bv[g * 16 + e, pl.ds(0, 16)]
                for j in range(1, H // 16):
                    v = v + (av[g * 16 + e, pl.ds(j * 16, 16)]
                             * bv[g * 16 + e, pl.ds(j * 16, 16)])
                ps[pl.ds(e * 16, 16)] = v
            res = plsc.load_gather(ps, [col_idx[0]])
            for l in range(1, 16):
                res = res + plsc.load_gather(ps, [col_idx[l]])
            ol[pl.ds(i * CL + g * 16, 16)] = res
            return 0
        lax.fori_loop(0, CL // 16, grp, 0)

    NPAIR = (LCH - 1) // 2
    fire(0, bufs_a, sem_a, True)

    def pairb(i, _):
        i0 = 2 * i
        fire(i0 + 1, bufs_b, sem_b, True)
        fire(i0, bufs_a, sem_a, False)
        compute(i0, bufs_a)
        fire(i0 + 2, bufs_a, sem_a, True)
        fire(i0 + 1, bufs_b, sem_b, False)
        compute(i0 + 1, bufs_b)
        return 0
    lax.fori_loop(0, NPAIR, pairb, 0)
    fire(LCH - 1, bufs_a, sem_a, False)
    compute(LCH - 1, bufs_a)
    pltpu.sync_copy(ol, out.at[pl.ds(base, LCH * CL)])


_classify = pl.kernel(
    _classify_body,
    out_type=jax.ShapeDtypeStruct((E,), jnp.float32),
    mesh=_mesh,
    scratch_types=[
        pltpu.VMEM((LCH, CL), jnp.int32),
        pltpu.VMEM((LCH, CL), jnp.int32),
        [pltpu.VMEM((CL, H), jnp.float32)] * 2,
        [pltpu.VMEM((CL, H), jnp.float32)] * 2,
        pltpu.VMEM((256,), jnp.float32),
        pltpu.VMEM((LCH * CL,), jnp.float32),
        pltpu.SemaphoreType.DMA,
        pltpu.SemaphoreType.DMA,
    ],
    compiler_params=_sc_params,
)


def _tc_layer1(mean, xg, xd, wl, wr, b):
    BLK = 2000
    NBLK = N // BLK

    def body(mean_ref, xg_ref, xd_ref, wl_ref, wr_ref, b_ref, olo, ohi):
        g = pl.program_id(0)
        x = jnp.where(g == 0, xg_ref[...], xd_ref[...])
        acc = jnp.dot(mean_ref[0], wl_ref[0],
                      preferred_element_type=jnp.float32)
        acc += jnp.dot(x, wr_ref[0], preferred_element_type=jnp.float32)
        acc += b_ref[0]
        acc = jnp.maximum(acc, 0.0)
        olo[0] = acc[:, :H2]
        ohi[0] = acc[:, H2:]

    wspec = pl.BlockSpec((1, H, H), lambda g, i: (g, 0, 0))
    xspec = pl.BlockSpec((BLK, H), lambda g, i: (i, 0))
    ohalf = pl.BlockSpec((1, BLK, H2), lambda g, i: (g, i, 0))
    return pl.pallas_call(
        body,
        grid=(2, NBLK),
        in_specs=[pl.BlockSpec((1, BLK, H), lambda g, i: (1 - g, i, 0)),
                  xspec, xspec, wspec, wspec,
                  pl.BlockSpec((1, 1, H), lambda g, i: (g, 0, 0))],
        out_specs=[ohalf, ohalf],
        out_shape=[jax.ShapeDtypeStruct((NC, NP, H2), jnp.float32),
                   jax.ShapeDtypeStruct((NC, NP, H2), jnp.float32)],
    )(mean, xg, xd, wl, wr, b)


def _tc_layer2(mean, h_lo, h_hi, wl, wr, b):
    BLK = 2000
    NBLK = N // BLK

    def body(mean_ref, xlo_ref, xhi_ref, wl_ref, wr_ref, b_ref, o_ref):
        acc = jnp.dot(mean_ref[0], wl_ref[0],
                      preferred_element_type=jnp.float32)
        acc += jnp.dot(xlo_ref[0], wr_ref[0, :H2, :],
                       preferred_element_type=jnp.float32)
        acc += jnp.dot(xhi_ref[0], wr_ref[0, H2:, :],
                       preferred_element_type=jnp.float32)
        acc += b_ref[0]
        o_ref[0] = acc

    wspec = pl.BlockSpec((1, H, H), lambda g, i: (g, 0, 0))
    xhalf = pl.BlockSpec((1, BLK, H2), lambda g, i: (g, i, 0))
    return pl.pallas_call(
        body,
        grid=(2, NBLK),
        in_specs=[pl.BlockSpec((1, BLK, H), lambda g, i: (1 - g, i, 0)),
                  xhalf, xhalf, wspec, wspec,
                  pl.BlockSpec((1, 1, H), lambda g, i: (g, 0, 0))],
        out_specs=pl.BlockSpec((1, BLK, H), lambda g, i: (g, i, 0)),
        out_shape=jax.ShapeDtypeStruct((NC, NP, H), jnp.float32),
    )(mean, h_lo, h_hi, wl, wr, b)


def kernel(gene_node_id, disease_node_id, edge_index_gda, edge_index_rev,
           edge_label_index, gene_emb, disease_emb,
           W1_gda_l, W1_gda_r, b1_gda, W1_rev_l, W1_rev_r, b1_rev,
           W2_gda_l, W2_gda_r, b2_gda, W2_rev_l, W2_rev_r, b2_rev):
    f32, i32 = jnp.float32, jnp.int32
    gene_emb = gene_emb.astype(f32)
    disease_emb = disease_emb.astype(f32)
    g_lo, g_hi = gene_emb[:, :H2], gene_emb[:, H2:]
    d_lo, d_hi = disease_emb[:, :H2], disease_emb[:, H2:]

    eg = edge_index_gda.astype(i32).reshape(2, NS, NCHUNK, CE)
    er = edge_index_rev.astype(i32).reshape(2, NS, NCHUNK, CE)
    el = edge_label_index.astype(i32).reshape(2, NC, NS, LCH, CL)

    dummy_inv = jnp.zeros((NC, NP), f32)
    m1, inv = _agg_first(g_lo, g_hi, d_lo, d_hi, eg, er, dummy_inv)

    wl1 = jnp.stack([W1_rev_l, W1_gda_l])
    wr1 = jnp.stack([W1_rev_r, W1_gda_r])
    bb1 = jnp.stack([b1_rev, b1_gda]).reshape(NC, 1, H)
    ht_lo, ht_hi = _tc_layer1(m1, gene_emb, disease_emb, wl1, wr1, bb1)

    m2, _ = _agg_more(ht_lo[0], ht_hi[0], ht_lo[1], ht_hi[1], eg, er, inv)

    wl2 = jnp.stack([W2_rev_l, W2_gda_l])
    wr2 = jnp.stack([W2_rev_r, W2_gda_r])
    bb2 = jnp.stack([b2_rev, b2_gda]).reshape(NC, 1, H)
    zt = _tc_layer2(m2, ht_lo, ht_hi, wl2, wr2, bb2)

    return _classify(zt[0], zt[1], el)
